# Initial kernel scaffold; baseline (speedup 1.0000x reference)
import jax
import jax.numpy as jnp
from jax.experimental import pallas as pl


def kernel(x, edge_index, W1, b1, W2, b2, Wb, bb, perm):
    raise NotImplementedError("write your pallas kernel here")


# speedup vs baseline: 12.3938x; 12.3938x over previous
import functools

import jax
import jax.numpy as jnp
from jax import lax
from jax.experimental import pallas as pl
from jax.experimental.pallas import tpu as pltpu
from jax.experimental.pallas import tpu_sc as plsc

_N = 10000
_E = 320000
_D = 128
_CH = 128
_NCHUNK = _E // _CH
_NSUB = 16
_NCORE = 2
_NPAD = 10240
_RPS = _N // _NSUB

_f32 = jnp.float32


def _sc_mesh():
    return plsc.VectorSubcoreMesh(core_axis_name="c", subcore_axis_name="s")


def _deg_kernel(dst_hbm, out_a, out_b, acc_sh, idx_v, ones_v, zero_v):
    c = lax.axis_index("c")
    s = lax.axis_index("s")

    ones = jnp.full((16,), 1.0, dtype=_f32)
    for i in range(8):
        ones_v[pl.ds(i * 16, 16)] = ones
    zeros = jnp.zeros((16,), dtype=_f32)
    for i in range(40):
        zero_v[pl.ds(i * 16, 16)] = zeros
    pltpu.sync_copy(zero_v, acc_sh.at[pl.ds(s * 640, 640)])
    plsc.subcore_barrier()

    half = _NCHUNK // _NCORE

    def body(j, carry):
        lid = j * _NSUB + s

        @pl.when(lid < half)
        def _():
            cid = c * half + lid
            pltpu.sync_copy(dst_hbm.at[pl.ds(cid * _CH, _CH)], idx_v)
            pltpu.sync_copy(ones_v, acc_sh.at[idx_v], add=True)

        return carry

    lax.fori_loop(0, (half + _NSUB - 1) // _NSUB, body, 0)
    plsc.subcore_barrier()

    @pl.when(c == 0)
    def _():
        pltpu.sync_copy(acc_sh.at[pl.ds(s * 640, 640)], out_a.at[pl.ds(s * 640, 640)])

    @pl.when(c == 1)
    def _():
        pltpu.sync_copy(acc_sh.at[pl.ds(s * 640, 640)], out_b.at[pl.ds(s * 640, 640)])


def _sc_deg(dst):
    kern = pl.kernel(
        _deg_kernel,
        mesh=_sc_mesh(),
        out_type=(
            jax.ShapeDtypeStruct((_NPAD,), _f32),
            jax.ShapeDtypeStruct((_NPAD,), _f32),
        ),
        scratch_types=[
            pltpu.VMEM_SHARED((_NPAD,), _f32),
            pltpu.VMEM((_CH,), jnp.int32),
            pltpu.VMEM((_CH,), _f32),
            pltpu.VMEM((640,), _f32),
        ],
    )
    return kern(dst)


def _perm_kernel(h1_hbm, perm_hbm, out_hbm, idx_v, rows_v, idx_t, rows_t, sem):
    c = lax.axis_index("c")
    s = lax.axis_index("s")
    w = s * _NCORE + c

    nfull = _N // _CH
    nw = _NCORE * _NSUB

    def body(j, carry):
        cid = j * nw + w

        @pl.when(cid < nfull)
        def _():
            pltpu.sync_copy(perm_hbm.at[pl.ds(cid * _CH, _CH)], idx_v)
            pltpu.async_copy(h1_hbm.at[idx_v], rows_v, sem).wait()
            pltpu.sync_copy(rows_v, out_hbm.at[pl.ds(cid * _CH, _CH)])

        return carry

    lax.fori_loop(0, (nfull + nw - 1) // nw, body, 0)

    tail = _N - nfull * _CH

    @pl.when(w == 0)
    def _():
        pltpu.sync_copy(perm_hbm.at[pl.ds(nfull * _CH, tail)], idx_t)
        pltpu.async_copy(h1_hbm.at[idx_t], rows_t, sem).wait()
        pltpu.sync_copy(rows_t, out_hbm.at[pl.ds(nfull * _CH, tail)])


def _sc_perm(h1, perm):
    kern = pl.kernel(
        _perm_kernel,
        mesh=_sc_mesh(),
        out_type=jax.ShapeDtypeStruct((_N, _D), _f32),
        scratch_types=[
            pltpu.VMEM((_CH,), jnp.int32),
            pltpu.VMEM((_CH, _D), _f32),
            pltpu.VMEM((16,), jnp.int32),
            pltpu.VMEM((16, _D), _f32),
            pltpu.SemaphoreType.DMA,
        ],
    )
    return kern(h1, perm)


_RCH = 80
_NRCH = _N // _RCH


def _agg_half(tbl, out, src_hbm, dst_hbm, acc_sh, idx_s, idx_d, rows, sem, s):
    def init_body(j, carry):
        cid = j * _NSUB + s

        @pl.when(cid < _NRCH)
        def _():
            pltpu.sync_copy(tbl.at[pl.ds(cid * _RCH, _RCH)],
                            acc_sh.at[pl.ds(cid * _RCH, _RCH)])

        return carry

    lax.fori_loop(0, (_NRCH + _NSUB - 1) // _NSUB, init_body, 0)
    plsc.subcore_barrier()

    def body(j, carry):
        cid = j * _NSUB + s

        @pl.when(cid < _NCHUNK)
        def _():
            pltpu.sync_copy(src_hbm.at[pl.ds(cid * _CH, _CH)], idx_s)
            pltpu.async_copy(tbl.at[idx_s], rows, sem).wait()
            pltpu.sync_copy(dst_hbm.at[pl.ds(cid * _CH, _CH)], idx_d)
            pltpu.sync_copy(rows, acc_sh.at[idx_d], add=True)

        return carry

    lax.fori_loop(0, (_NCHUNK + _NSUB - 1) // _NSUB, body, 0)
    plsc.subcore_barrier()

    def wb_body(j, carry):
        cid = j * _NSUB + s

        @pl.when(cid < _NRCH)
        def _():
            pltpu.sync_copy(acc_sh.at[pl.ds(cid * _RCH, _RCH)],
                            out.at[pl.ds(cid * _RCH, _RCH)])

        return carry

    lax.fori_loop(0, (_NRCH + _NSUB - 1) // _NSUB, wb_body, 0)


def _agg_kernel(tbl_a, tbl_b, src_hbm, dst_hbm, out_a, out_b,
                acc_sh, idx_s, idx_d, rows, sem):
    c = lax.axis_index("c")
    s = lax.axis_index("s")

    @pl.when(c == 0)
    def _():
        _agg_half(tbl_a, out_a, src_hbm, dst_hbm, acc_sh, idx_s, idx_d, rows, sem, s)

    @pl.when(c == 1)
    def _():
        _agg_half(tbl_b, out_b, src_hbm, dst_hbm, acc_sh, idx_s, idx_d, rows, sem, s)


def _sc_agg(tbl_a, tbl_b, src, dst):
    kern = pl.kernel(
        _agg_kernel,
        mesh=_sc_mesh(),
        out_type=(
            jax.ShapeDtypeStruct((_N, _D), _f32),
            jax.ShapeDtypeStruct((_N, _D), _f32),
        ),
        scratch_types=[
            pltpu.VMEM_SHARED((_N, _D), _f32),
            pltpu.VMEM((_CH,), jnp.int32),
            pltpu.VMEM((_CH,), jnp.int32),
            pltpu.VMEM((_CH, _D), _f32),
            pltpu.SemaphoreType.DMA,
        ],
    )
    return kern(tbl_a, tbl_b, src, dst)


_BR = 1000


def _dinv(pa, pb):
    return lax.rsqrt(pa + pb + 1.0)


def _mm_body(x_ref, w_ref, o_ref):
    o_ref[...] = jnp.dot(x_ref[...], w_ref[...], preferred_element_type=_f32)


def _tc_matmul(x, W):
    return pl.pallas_call(
        _mm_body,
        grid=(_N // _BR,),
        in_specs=[
            pl.BlockSpec((_BR, _D), lambda i: (i, 0)),
            pl.BlockSpec((_D, _D), lambda i: (0, 0)),
        ],
        out_specs=pl.BlockSpec((_BR, _D), lambda i: (i, 0)),
        out_shape=jax.ShapeDtypeStruct((_N, _D), _f32),
    )(x, W)


def _scale2_body(h_ref, hp_ref, pa_ref, pb_ref, oa_ref, ob_ref):
    d = _dinv(pa_ref[...], pb_ref[...])
    oa_ref[...] = h_ref[...] * d
    ob_ref[...] = hp_ref[...] * d


def _tc_scale2(h1, h1p, pa, pb):
    return pl.pallas_call(
        _scale2_body,
        grid=(_N // _BR,),
        in_specs=[
            pl.BlockSpec((_BR, _D), lambda i: (i, 0)),
            pl.BlockSpec((_BR, _D), lambda i: (i, 0)),
            pl.BlockSpec((_BR, 1), lambda i: (i, 0)),
            pl.BlockSpec((_BR, 1), lambda i: (i, 0)),
        ],
        out_specs=[
            pl.BlockSpec((_BR, _D), lambda i: (i, 0)),
            pl.BlockSpec((_BR, _D), lambda i: (i, 0)),
        ],
        out_shape=[
            jax.ShapeDtypeStruct((_N, _D), _f32),
            jax.ShapeDtypeStruct((_N, _D), _f32),
        ],
    )(h1, h1p, pa, pb)


def _layer2_body(sa_ref, sb_ref, pa_ref, pb_ref, b1_ref, w2_ref, oa_ref, ob_ref):
    d = _dinv(pa_ref[...], pb_ref[...])
    za = jnp.maximum(sa_ref[...] * d + b1_ref[...], 0.0)
    zb = jnp.maximum(sb_ref[...] * d + b1_ref[...], 0.0)
    oa_ref[...] = jnp.dot(za, w2_ref[...], preferred_element_type=_f32) * d
    ob_ref[...] = jnp.dot(zb, w2_ref[...], preferred_element_type=_f32) * d


def _tc_layer2(sa, sb, pa, pb, b1, W2):
    return pl.pallas_call(
        _layer2_body,
        grid=(_N // _BR,),
        in_specs=[
            pl.BlockSpec((_BR, _D), lambda i: (i, 0)),
            pl.BlockSpec((_BR, _D), lambda i: (i, 0)),
            pl.BlockSpec((_BR, 1), lambda i: (i, 0)),
            pl.BlockSpec((_BR, 1), lambda i: (i, 0)),
            pl.BlockSpec((1, _D), lambda i: (0, 0)),
            pl.BlockSpec((_D, _D), lambda i: (0, 0)),
        ],
        out_specs=[
            pl.BlockSpec((_BR, _D), lambda i: (i, 0)),
            pl.BlockSpec((_BR, _D), lambda i: (i, 0)),
        ],
        out_shape=[
            jax.ShapeDtypeStruct((_N, _D), _f32),
            jax.ShapeDtypeStruct((_N, _D), _f32),
        ],
    )(sa, sb, pa, pb, b1, W2)


def _readout_body(sa_ref, sb_ref, pa_ref, pb_ref, b2_ref, wb_ref, bb_ref,
                  pos_ref, neg_ref, colsum, vrow, c0):
    p = pl.program_id(0)
    j = pl.program_id(1)
    d = _dinv(pa_ref[...], pb_ref[...])

    @pl.when(p == 0)
    def _():
        @pl.when(j == 0)
        def _():
            colsum[...] = jnp.zeros_like(colsum)

        ha = sa_ref[...] * d
        colsum[...] += jnp.sum(ha, axis=0, keepdims=True)

    @pl.when(p == 1)
    def _():
        @pl.when(j == 0)
        def _():
            mean = colsum[...] * (1.0 / _N) + b2_ref[...]
            srow = 1.0 / (1.0 + jnp.exp(-mean))
            v = lax.dot_general(srow, wb_ref[...], (((1,), (1,)), ((), ())),
                                preferred_element_type=_f32)
            vrow[...] = v
            c0[...] = jnp.sum(b2_ref[...] * v, axis=1, keepdims=True) + bb_ref[...]

        ha = sa_ref[...] * d
        hb = sb_ref[...] * d
        v = vrow[...]
        pos_ref[...] = jnp.sum(ha * v, axis=1, keepdims=True) + c0[...]
        neg_ref[...] = jnp.sum(hb * v, axis=1, keepdims=True) + c0[...]


def _tc_readout(sa, sb, pa, pb, b2, wb, bb):
    return pl.pallas_call(
        _readout_body,
        grid=(2, _N // _BR),
        in_specs=[
            pl.BlockSpec((_BR, _D), lambda p, j: (j, 0)),
            pl.BlockSpec((_BR, _D), lambda p, j: (j, 0)),
            pl.BlockSpec((_BR, 1), lambda p, j: (j, 0)),
            pl.BlockSpec((_BR, 1), lambda p, j: (j, 0)),
            pl.BlockSpec((1, _D), lambda p, j: (0, 0)),
            pl.BlockSpec((_D, _D), lambda p, j: (0, 0)),
            pl.BlockSpec((1, 1), lambda p, j: (0, 0)),
        ],
        out_specs=[
            pl.BlockSpec((_BR, 1), lambda p, j: (j, 0)),
            pl.BlockSpec((_BR, 1), lambda p, j: (j, 0)),
        ],
        out_shape=[
            jax.ShapeDtypeStruct((_N, 1), _f32),
            jax.ShapeDtypeStruct((_N, 1), _f32),
        ],
        scratch_shapes=[
            pltpu.VMEM((1, _D), _f32),
            pltpu.VMEM((1, _D), _f32),
            pltpu.VMEM((1, 1), _f32),
        ],
    )(sa, sb, pa, pb, b2, wb, bb)


def kernel(x, edge_index, W1, b1, W2, b2, Wb, bb, perm):
    src = edge_index[0]
    dst = edge_index[1]
    perm = perm.astype(jnp.int32)

    dega, degb = _sc_deg(dst)
    pa = dega[:_N].reshape(_N, 1)
    pb = degb[:_N].reshape(_N, 1)

    h1 = _tc_matmul(x, W1)
    h1p = _sc_perm(h1, perm)
    p1a, p1b = _tc_scale2(h1, h1p, pa, pb)
    s1a, s1b = _sc_agg(p1a, p1b, src, dst)

    p2a, p2b = _tc_layer2(s1a, s1b, pa, pb, b1.reshape(1, _D), W2)
    s2a, s2b = _sc_agg(p2a, p2b, src, dst)

    pos, neg = _tc_readout(s2a, s2b, pa, pb, b2.reshape(1, _D),
                           Wb.reshape(_D, _D), bb.reshape(1, 1))
    return (pos, neg)
